# baseline (speedup 1.0000x reference)
import jax
import jax.numpy as jnp
from jax import lax
from jax.experimental import pallas as pl
from jax.experimental.pallas import tpu as pltpu
from jax.experimental.pallas import tpu_sc as plsc

_N = 10000
_K = 32
_D = 128

_BLOCK = 400
_N_SC = 2560
_SC_START = _N - _N_SC
_MAIN_BLOCKS = 19
_NW = 32
_RPW = _N_SC // _NW
_CH = 2
_NCHUNK = _RPW // _CH
_FBLOCK = 80


def _fused_body(src_ref, nbr_ref, w_ref, wa_ref, out_ref):
    agg = jnp.sum(nbr_ref[...], axis=1) * (1.0 / _K)
    self_h = jnp.dot(src_ref[...], w_ref[...], preferred_element_type=jnp.float32)
    nbr_h = jnp.dot(agg, wa_ref[...], preferred_element_type=jnp.float32)
    out_ref[...] = jnp.maximum(self_h + nbr_h, 0.0)


def _finish_body(main_ref, src_ref, sums_ref, w_ref, wa_ref, out_ref):
    agg = sums_ref[...] * (1.0 / _K)
    self_h = jnp.dot(src_ref[...], w_ref[...], preferred_element_type=jnp.float32)
    nbr_h = jnp.dot(agg, wa_ref[...], preferred_element_type=jnp.float32)
    out_ref[...] = jnp.maximum(self_h + nbr_h, 0.0)


def _sc_sum_body(nbr_hbm, out_hbm, slab0, slab1, acc_v, sem0, sem1):
    wid = lax.axis_index("s") * 2 + lax.axis_index("c")
    base = (_SC_START + wid * _RPW) * _K
    fpc = _CH * _K

    slabs = (slab0, slab1)
    sems = (sem0, sem1)

    def start(chunk, buf):
        pltpu.make_async_copy(
            nbr_hbm.at[pl.ds(base + chunk * fpc, fpc)], slabs[buf], sems[buf]
        ).start()

    def wait(buf):
        pltpu.make_async_copy(
            nbr_hbm.at[pl.ds(base, fpc)], slabs[buf], sems[buf]
        ).wait()

    start(0, 0)
    start(1, 1)

    def loop_body(i, _):
        c2 = i * 2
        for b in range(2):
            wait(b)
            slab = slabs[b]
            for j in range(_CH):
                accs = [jnp.zeros((16,), jnp.float32) for _ in range(_D // 16)]
                for k in range(_K):
                    fr = j * _K + k
                    for d in range(_D // 16):
                        accs[d] = accs[d] + slab[fr, pl.ds(d * 16, 16)]
                row = (c2 + b) * _CH + j
                for d in range(_D // 16):
                    acc_v[row, pl.ds(d * 16, 16)] = accs[d]
            nxt = c2 + 2 + b

            @pl.when(nxt < _NCHUNK)
            def _():
                start(nxt, b)

        return 0

    lax.fori_loop(0, _NCHUNK // 2, loop_body, 0)
    pltpu.sync_copy(acc_v, out_hbm.at[pl.ds(wid * _RPW, _RPW)])


def _sc_neighbor_sums(nbr_flat):
    mesh = plsc.VectorSubcoreMesh(core_axis_name="c", subcore_axis_name="s")
    return pl.kernel(
        _sc_sum_body,
        mesh=mesh,
        out_type=jax.ShapeDtypeStruct((_N_SC, _D), jnp.float32),
        scratch_types=[
            pltpu.VMEM((_CH * _K, _D), jnp.float32),
            pltpu.VMEM((_CH * _K, _D), jnp.float32),
            pltpu.VMEM((_RPW, _D), jnp.float32),
            pltpu.SemaphoreType.DMA,
            pltpu.SemaphoreType.DMA,
        ],
    )(nbr_flat)


def kernel(src_node_features, neighbor_node_features, weight, agg_weight):
    n, d = src_node_features.shape
    k = neighbor_node_features.shape[1]
    nbr_flat = neighbor_node_features.reshape(n * k, d)

    main_out = pl.pallas_call(
        _fused_body,
        grid=(_MAIN_BLOCKS,),
        in_specs=[
            pl.BlockSpec((_BLOCK, d), lambda i: (i, 0)),
            pl.BlockSpec((_BLOCK, k, d), lambda i: (i, 0, 0)),
            pl.BlockSpec((d, d), lambda i: (0, 0)),
            pl.BlockSpec((d, d), lambda i: (0, 0)),
        ],
        out_specs=pl.BlockSpec((_BLOCK, d), lambda i: (i, 0)),
        out_shape=jax.ShapeDtypeStruct((n, d), jnp.float32),
    )(src_node_features, neighbor_node_features, weight, agg_weight)

    sc_sums = _sc_neighbor_sums(nbr_flat)

    off = _SC_START // _FBLOCK
    return pl.pallas_call(
        _finish_body,
        grid=(_N_SC // _FBLOCK,),
        in_specs=[
            pl.BlockSpec((_FBLOCK, d), lambda i: (i + off, 0)),
            pl.BlockSpec((_FBLOCK, d), lambda i: (i + off, 0)),
            pl.BlockSpec((_FBLOCK, d), lambda i: (i, 0)),
            pl.BlockSpec((d, d), lambda i: (0, 0)),
            pl.BlockSpec((d, d), lambda i: (0, 0)),
        ],
        out_specs=pl.BlockSpec((_FBLOCK, d), lambda i: (i + off, 0)),
        out_shape=jax.ShapeDtypeStruct((n, d), jnp.float32),
        input_output_aliases={0: 0},
    )(main_out, src_node_features, sc_sums, weight, agg_weight)


# speedup vs baseline: 1.0080x; 1.0080x over previous
import jax
import jax.numpy as jnp
from jax import lax
from jax.experimental import pallas as pl
from jax.experimental.pallas import tpu as pltpu
from jax.experimental.pallas import tpu_sc as plsc

_N = 10000
_K = 32
_D = 128

_BLOCK = 400
_N_SC = 2560
_SC_START = _N - _N_SC
_MAIN_BLOCKS = 19
_NW = 32
_RPW = _N_SC // _NW
_CH = 4
_NCHUNK = _RPW // _CH
_FBLOCK = 80


def _fused_body(src_ref, nbr_ref, w_ref, wa_ref, out_ref):
    agg = jnp.sum(nbr_ref[...], axis=1) * (1.0 / _K)
    self_h = jnp.dot(src_ref[...], w_ref[...], preferred_element_type=jnp.float32)
    nbr_h = jnp.dot(agg, wa_ref[...], preferred_element_type=jnp.float32)
    out_ref[...] = jnp.maximum(self_h + nbr_h, 0.0)


def _finish_body(main_ref, src_ref, sums_ref, w_ref, wa_ref, out_ref):
    agg = sums_ref[...] * (1.0 / _K)
    self_h = jnp.dot(src_ref[...], w_ref[...], preferred_element_type=jnp.float32)
    nbr_h = jnp.dot(agg, wa_ref[...], preferred_element_type=jnp.float32)
    out_ref[...] = jnp.maximum(self_h + nbr_h, 0.0)


def _sc_sum_body(nbr_hbm, out_hbm, slab0, slab1, acc_v, sem0, sem1):
    wid = lax.axis_index("s") * 2 + lax.axis_index("c")
    base = (_SC_START + wid * _RPW) * _K
    fpc = _CH * _K

    slabs = (slab0, slab1)
    sems = (sem0, sem1)

    def start(chunk, buf):
        pltpu.make_async_copy(
            nbr_hbm.at[pl.ds(base + chunk * fpc, fpc)], slabs[buf], sems[buf]
        ).start()

    def wait(buf):
        pltpu.make_async_copy(
            nbr_hbm.at[pl.ds(base, fpc)], slabs[buf], sems[buf]
        ).wait()

    start(0, 0)
    start(1, 1)

    def loop_body(i, _):
        c2 = i * 2
        for b in range(2):
            wait(b)
            slab = slabs[b]
            for j in range(_CH):
                accs = [jnp.zeros((16,), jnp.float32) for _ in range(_D // 16)]
                for k in range(_K):
                    fr = j * _K + k
                    for d in range(_D // 16):
                        accs[d] = accs[d] + slab[fr, pl.ds(d * 16, 16)]
                row = (c2 + b) * _CH + j
                for d in range(_D // 16):
                    acc_v[row, pl.ds(d * 16, 16)] = accs[d]
            nxt = c2 + 2 + b

            @pl.when(nxt < _NCHUNK)
            def _():
                start(nxt, b)

        return 0

    lax.fori_loop(0, _NCHUNK // 2, loop_body, 0)
    pltpu.sync_copy(acc_v, out_hbm.at[pl.ds(wid * _RPW, _RPW)])


def _sc_neighbor_sums(nbr_flat):
    mesh = plsc.VectorSubcoreMesh(core_axis_name="c", subcore_axis_name="s")
    return pl.kernel(
        _sc_sum_body,
        mesh=mesh,
        out_type=jax.ShapeDtypeStruct((_N_SC, _D), jnp.float32),
        scratch_types=[
            pltpu.VMEM((_CH * _K, _D), jnp.float32),
            pltpu.VMEM((_CH * _K, _D), jnp.float32),
            pltpu.VMEM((_RPW, _D), jnp.float32),
            pltpu.SemaphoreType.DMA,
            pltpu.SemaphoreType.DMA,
        ],
    )(nbr_flat)


def kernel(src_node_features, neighbor_node_features, weight, agg_weight):
    n, d = src_node_features.shape
    k = neighbor_node_features.shape[1]
    nbr_flat = neighbor_node_features.reshape(n * k, d)

    sc_sums = _sc_neighbor_sums(nbr_flat)

    main_out = pl.pallas_call(
        _fused_body,
        grid=(_MAIN_BLOCKS,),
        in_specs=[
            pl.BlockSpec((_BLOCK, d), lambda i: (i, 0)),
            pl.BlockSpec((_BLOCK, k, d), lambda i: (i, 0, 0)),
            pl.BlockSpec((d, d), lambda i: (0, 0)),
            pl.BlockSpec((d, d), lambda i: (0, 0)),
        ],
        out_specs=pl.BlockSpec((_BLOCK, d), lambda i: (i, 0)),
        out_shape=jax.ShapeDtypeStruct((n, d), jnp.float32),
    )(src_node_features, neighbor_node_features, weight, agg_weight)

    off = _SC_START // _FBLOCK
    return pl.pallas_call(
        _finish_body,
        grid=(_N_SC // _FBLOCK,),
        in_specs=[
            pl.BlockSpec((_FBLOCK, d), lambda i: (i + off, 0)),
            pl.BlockSpec((_FBLOCK, d), lambda i: (i + off, 0)),
            pl.BlockSpec((_FBLOCK, d), lambda i: (i, 0)),
            pl.BlockSpec((d, d), lambda i: (0, 0)),
            pl.BlockSpec((d, d), lambda i: (0, 0)),
        ],
        out_specs=pl.BlockSpec((_FBLOCK, d), lambda i: (i + off, 0)),
        out_shape=jax.ShapeDtypeStruct((n, d), jnp.float32),
        input_output_aliases={0: 0},
    )(main_out, src_node_features, sc_sums, weight, agg_weight)


# speedup vs baseline: 1.2727x; 1.2626x over previous
import jax
import jax.numpy as jnp
from jax import lax
from jax.experimental import pallas as pl
from jax.experimental.pallas import tpu as pltpu
from jax.experimental.pallas import tpu_sc as plsc

_N = 10000
_K = 32
_D = 128

_BLOCK = 400
_N_SC = 1600
_SC_START = _N - _N_SC
_MAIN_BLOCKS = _SC_START // _BLOCK
_RPW = 80
_NW_ACTIVE = _N_SC // _RPW
_CH = 4
_NCHUNK = _RPW // _CH


def _fused_body(src_ref, nbr_ref, w_ref, wa_ref, out_ref):
    agg = jnp.sum(nbr_ref[...], axis=1) * (1.0 / _K)
    self_h = jnp.dot(src_ref[...], w_ref[...], preferred_element_type=jnp.float32)
    nbr_h = jnp.dot(agg, wa_ref[...], preferred_element_type=jnp.float32)
    out_ref[...] = jnp.maximum(self_h + nbr_h, 0.0)


def _finish_body(main_ref, src_ref, sums_ref, w_ref, wa_ref, out_ref):
    agg = sums_ref[...] * (1.0 / _K)
    self_h = jnp.dot(src_ref[...], w_ref[...], preferred_element_type=jnp.float32)
    nbr_h = jnp.dot(agg, wa_ref[...], preferred_element_type=jnp.float32)
    out_ref[...] = jnp.maximum(self_h + nbr_h, 0.0)


def _sc_sum_body(nbr_hbm, out_hbm, slab0, slab1, acc_v, sem0, sem1):
    wid = lax.axis_index("s") * 2 + lax.axis_index("c")
    base = (_SC_START + wid * _RPW) * _K
    fpc = _CH * _K

    slabs = (slab0, slab1)
    sems = (sem0, sem1)

    @pl.when(wid < _NW_ACTIVE)
    def _active():
        _sc_worker(nbr_hbm, out_hbm, slabs, sems, acc_v, base, wid)


def _sc_worker(nbr_hbm, out_hbm, slabs, sems, acc_v, base, wid):
    fpc = _CH * _K

    def start(chunk, buf):
        pltpu.make_async_copy(
            nbr_hbm.at[pl.ds(base + chunk * fpc, fpc)], slabs[buf], sems[buf]
        ).start()

    def wait(buf):
        pltpu.make_async_copy(
            nbr_hbm.at[pl.ds(base, fpc)], slabs[buf], sems[buf]
        ).wait()

    start(0, 0)
    start(1, 1)

    def loop_body(i, _):
        c2 = i * 2
        for b in range(2):
            wait(b)
            slab = slabs[b]
            for j in range(_CH):
                accs = [jnp.zeros((16,), jnp.float32) for _ in range(_D // 16)]
                for k in range(_K):
                    fr = j * _K + k
                    for d in range(_D // 16):
                        accs[d] = accs[d] + slab[fr, pl.ds(d * 16, 16)]
                row = (c2 + b) * _CH + j
                for d in range(_D // 16):
                    acc_v[row, pl.ds(d * 16, 16)] = accs[d]
            nxt = c2 + 2 + b

            @pl.when(nxt < _NCHUNK)
            def _():
                start(nxt, b)

        return 0

    lax.fori_loop(0, _NCHUNK // 2, loop_body, 0)
    pltpu.sync_copy(acc_v, out_hbm.at[pl.ds(wid * _RPW, _RPW)])


def _sc_neighbor_sums(nbr_flat):
    mesh = plsc.VectorSubcoreMesh(core_axis_name="c", subcore_axis_name="s")
    return pl.kernel(
        _sc_sum_body,
        mesh=mesh,
        out_type=jax.ShapeDtypeStruct((_N_SC, _D), jnp.float32),
        scratch_types=[
            pltpu.VMEM((_CH * _K, _D), jnp.float32),
            pltpu.VMEM((_CH * _K, _D), jnp.float32),
            pltpu.VMEM((_RPW, _D), jnp.float32),
            pltpu.SemaphoreType.DMA,
            pltpu.SemaphoreType.DMA,
        ],
    )(nbr_flat)


def kernel(src_node_features, neighbor_node_features, weight, agg_weight):
    n, d = src_node_features.shape
    k = neighbor_node_features.shape[1]
    nbr_flat = neighbor_node_features.reshape(n * k, d)

    sc_sums = _sc_neighbor_sums(nbr_flat)

    main_out = pl.pallas_call(
        _fused_body,
        grid=(_MAIN_BLOCKS,),
        in_specs=[
            pl.BlockSpec((_BLOCK, d), lambda i: (i, 0)),
            pl.BlockSpec((_BLOCK, k, d), lambda i: (i, 0, 0)),
            pl.BlockSpec((d, d), lambda i: (0, 0)),
            pl.BlockSpec((d, d), lambda i: (0, 0)),
        ],
        out_specs=pl.BlockSpec((_BLOCK, d), lambda i: (i, 0)),
        out_shape=jax.ShapeDtypeStruct((n, d), jnp.float32),
    )(src_node_features, neighbor_node_features, weight, agg_weight)

    off = _SC_START // _BLOCK
    return pl.pallas_call(
        _finish_body,
        grid=(_N_SC // _BLOCK,),
        in_specs=[
            pl.BlockSpec((_BLOCK, d), lambda i: (i + off, 0)),
            pl.BlockSpec((_BLOCK, d), lambda i: (i + off, 0)),
            pl.BlockSpec((_BLOCK, d), lambda i: (i, 0)),
            pl.BlockSpec((d, d), lambda i: (0, 0)),
            pl.BlockSpec((d, d), lambda i: (0, 0)),
        ],
        out_specs=pl.BlockSpec((_BLOCK, d), lambda i: (i + off, 0)),
        out_shape=jax.ShapeDtypeStruct((n, d), jnp.float32),
        input_output_aliases={0: 0},
    )(main_out, src_node_features, sc_sums, weight, agg_weight)


# speedup vs baseline: 1.3725x; 1.0785x over previous
import jax
import jax.numpy as jnp
from jax import lax
from jax.experimental import pallas as pl
from jax.experimental.pallas import tpu as pltpu
from jax.experimental.pallas import tpu_sc as plsc

_N = 10000
_K = 32
_D = 128

_BLOCK = 400
_N_SC = 1600
_SC_START = _N - _N_SC
_MAIN_BLOCKS = _SC_START // _BLOCK
_RPW = 80
_NW_ACTIVE = _N_SC // _RPW
_CH = 5
_NCHUNK = _RPW // _CH
_NBUF = 4


def _fused_body(src_ref, nbr_ref, w_ref, wa_ref, out_ref):
    agg = jnp.sum(nbr_ref[...], axis=1) * (1.0 / _K)
    self_h = jnp.dot(src_ref[...], w_ref[...], preferred_element_type=jnp.float32)
    nbr_h = jnp.dot(agg, wa_ref[...], preferred_element_type=jnp.float32)
    out_ref[...] = jnp.maximum(self_h + nbr_h, 0.0)


def _finish_body(main_ref, src_ref, sums_ref, w_ref, wa_ref, out_ref):
    agg = sums_ref[...] * (1.0 / _K)
    self_h = jnp.dot(src_ref[...], w_ref[...], preferred_element_type=jnp.float32)
    nbr_h = jnp.dot(agg, wa_ref[...], preferred_element_type=jnp.float32)
    out_ref[...] = jnp.maximum(self_h + nbr_h, 0.0)


def _sc_sum_body(nbr_hbm, out_hbm, *scratch):
    slabs = scratch[:_NBUF]
    acc_v = scratch[_NBUF]
    sems = scratch[_NBUF + 1:]
    wid = lax.axis_index("s") * 2 + lax.axis_index("c")
    base = (_SC_START + wid * _RPW) * _K

    @pl.when(wid < _NW_ACTIVE)
    def _active():
        _sc_worker(nbr_hbm, out_hbm, slabs, sems, acc_v, base, wid)


def _sc_worker(nbr_hbm, out_hbm, slabs, sems, acc_v, base, wid):
    fpc = _CH * _K

    def start(chunk, buf):
        pltpu.make_async_copy(
            nbr_hbm.at[pl.ds(base + chunk * fpc, fpc)], slabs[buf], sems[buf]
        ).start()

    def wait(buf):
        pltpu.make_async_copy(
            nbr_hbm.at[pl.ds(base, fpc)], slabs[buf], sems[buf]
        ).wait()

    for b in range(_NBUF):
        start(b, b)

    def reduce_slab(slab, chunk):
        def row_body(j, _):
            fr0 = j * _K
            accs = [slab[fr0, pl.ds(d * 16, 16)] for d in range(_D // 16)]
            for k in range(1, _K):
                for d in range(_D // 16):
                    accs[d] = accs[d] + slab[fr0 + k, pl.ds(d * 16, 16)]
            row = chunk * _CH + j
            for d in range(_D // 16):
                acc_v[row, pl.ds(d * 16, 16)] = accs[d]
            return 0

        lax.fori_loop(0, _CH, row_body, 0)

    def loop_body(i, _):
        c0 = i * _NBUF
        for b in range(_NBUF):
            wait(b)
            reduce_slab(slabs[b], c0 + b)
            nxt = c0 + _NBUF + b

            @pl.when(nxt < _NCHUNK)
            def _():
                start(nxt, b)

        return 0

    lax.fori_loop(0, _NCHUNK // _NBUF, loop_body, 0)
    pltpu.sync_copy(acc_v, out_hbm.at[pl.ds(wid * _RPW, _RPW)])


def _sc_neighbor_sums(nbr_flat):
    mesh = plsc.VectorSubcoreMesh(core_axis_name="c", subcore_axis_name="s")
    return pl.kernel(
        _sc_sum_body,
        mesh=mesh,
        out_type=jax.ShapeDtypeStruct((_N_SC, _D), jnp.float32),
        scratch_types=(
            [pltpu.VMEM((_CH * _K, _D), jnp.float32) for _ in range(_NBUF)]
            + [pltpu.VMEM((_RPW, _D), jnp.float32)]
            + [pltpu.SemaphoreType.DMA for _ in range(_NBUF)]
        ),
    )(nbr_flat)


def kernel(src_node_features, neighbor_node_features, weight, agg_weight):
    n, d = src_node_features.shape
    k = neighbor_node_features.shape[1]
    nbr_flat = neighbor_node_features.reshape(n * k, d)

    sc_sums = _sc_neighbor_sums(nbr_flat)

    main_out = pl.pallas_call(
        _fused_body,
        grid=(_MAIN_BLOCKS,),
        in_specs=[
            pl.BlockSpec((_BLOCK, d), lambda i: (i, 0)),
            pl.BlockSpec((_BLOCK, k, d), lambda i: (i, 0, 0)),
            pl.BlockSpec((d, d), lambda i: (0, 0)),
            pl.BlockSpec((d, d), lambda i: (0, 0)),
        ],
        out_specs=pl.BlockSpec((_BLOCK, d), lambda i: (i, 0)),
        out_shape=jax.ShapeDtypeStruct((n, d), jnp.float32),
    )(src_node_features, neighbor_node_features, weight, agg_weight)

    off = _SC_START // _BLOCK
    return pl.pallas_call(
        _finish_body,
        grid=(_N_SC // _BLOCK,),
        in_specs=[
            pl.BlockSpec((_BLOCK, d), lambda i: (i + off, 0)),
            pl.BlockSpec((_BLOCK, d), lambda i: (i + off, 0)),
            pl.BlockSpec((_BLOCK, d), lambda i: (i, 0)),
            pl.BlockSpec((d, d), lambda i: (0, 0)),
            pl.BlockSpec((d, d), lambda i: (0, 0)),
        ],
        out_specs=pl.BlockSpec((_BLOCK, d), lambda i: (i + off, 0)),
        out_shape=jax.ShapeDtypeStruct((n, d), jnp.float32),
        input_output_aliases={0: 0},
    )(main_out, src_node_features, sc_sums, weight, agg_weight)


# speedup vs baseline: 1.8678x; 1.3609x over previous
import jax
import jax.numpy as jnp
from jax.experimental import pallas as pl
from jax.experimental.pallas import tpu as pltpu

_N = 10000
_K = 32
_D = 128
_BLOCK = 400


def _fused_body(src_ref, nbr_ref, w_ref, wa_ref, out_ref):
    agg = jnp.sum(nbr_ref[...], axis=1) * (1.0 / _K)
    self_h = jnp.dot(src_ref[...], w_ref[...], preferred_element_type=jnp.float32)
    nbr_h = jnp.dot(agg, wa_ref[...], preferred_element_type=jnp.float32)
    out_ref[...] = jnp.maximum(self_h + nbr_h, 0.0)


def kernel(src_node_features, neighbor_node_features, weight, agg_weight):
    n, d = src_node_features.shape
    k = neighbor_node_features.shape[1]
    grid = (pl.cdiv(n, _BLOCK),)
    return pl.pallas_call(
        _fused_body,
        grid=grid,
        in_specs=[
            pl.BlockSpec((_BLOCK, d), lambda i: (i, 0)),
            pl.BlockSpec((_BLOCK, k, d), lambda i: (i, 0, 0)),
            pl.BlockSpec((d, d), lambda i: (0, 0)),
            pl.BlockSpec((d, d), lambda i: (0, 0)),
        ],
        out_specs=pl.BlockSpec((_BLOCK, d), lambda i: (i, 0)),
        out_shape=jax.ShapeDtypeStruct((n, d), jnp.float32),
    )(src_node_features, neighbor_node_features, weight, agg_weight)


# speedup vs baseline: 1.8690x; 1.0006x over previous
import jax
import jax.numpy as jnp
from jax.experimental import pallas as pl

_K = 32
_BLOCK = 400


def _fused_body(src_ref, nbr_ref, w_ref, wa_ref, out_ref):
    agg = jnp.sum(nbr_ref[...], axis=1) * (1.0 / _K)
    self_h = jnp.dot(src_ref[...], w_ref[...], preferred_element_type=jnp.float32)
    nbr_h = jnp.dot(agg, wa_ref[...], preferred_element_type=jnp.float32)
    out_ref[...] = jnp.maximum(self_h + nbr_h, 0.0)


def kernel(src_node_features, neighbor_node_features, weight, agg_weight):
    n, d = src_node_features.shape
    k = neighbor_node_features.shape[1]
    grid = (pl.cdiv(n, _BLOCK),)
    return pl.pallas_call(
        _fused_body,
        grid=grid,
        in_specs=[
            pl.BlockSpec((_BLOCK, d), lambda i: (i, 0)),
            pl.BlockSpec((_BLOCK, k, d), lambda i: (i, 0, 0)),
            pl.BlockSpec((d, d), lambda i: (0, 0)),
            pl.BlockSpec((d, d), lambda i: (0, 0)),
        ],
        out_specs=pl.BlockSpec((_BLOCK, d), lambda i: (i, 0)),
        out_shape=jax.ShapeDtypeStruct((n, d), jnp.float32),
    )(src_node_features, neighbor_node_features, weight, agg_weight)
